# baseline (speedup 1.0000x reference)
import functools

import jax
import jax.numpy as jnp
from jax import lax
from jax.experimental import pallas as pl
from jax.experimental.pallas import tpu as pltpu
from jax.experimental.pallas import tpu_sc as plsc

_NW = 32
_L = 16
_WCH = 200


def _widen_kernel(n_chunks, table_hbm, wide_hbm,
                  vin0, vin1, vout0, vout1, si0, si1, so0, so1):
    wid = lax.axis_index("s") * 2 + lax.axis_index("c")
    max_t = (n_chunks + _NW - 1) // _NW
    vins = (vin0, vin1)
    vouts = (vout0, vout1)
    sis = (si0, si1)
    sos = (so0, so1)

    def stage_in(c, vin, sem):
        pltpu.async_copy(table_hbm.at[pl.ds(c * _WCH, _WCH)], vin, sem)

    def drain_in(vin, sem):
        pltpu.make_async_copy(
            table_hbm.at[pl.ds(0, _WCH)], vin, sem).wait()

    def fill(vin, vout):
        def rows(r2, _):
            r = r2 * 2
            for dr in range(2):
                for k in range(64 // _L):
                    vout[r + dr, pl.ds(k * _L, _L)] = (
                        vin[r + dr, pl.ds(k * _L, _L)])
            return 0

        lax.fori_loop(0, _WCH // 2, rows, 0)

    def stage_out(c, vout, sem):
        pltpu.async_copy(vout, wide_hbm.at[pl.ds(c * _WCH, _WCH)], sem)

    def drain_out(vout, sem):
        pltpu.make_async_copy(
            vout, wide_hbm.at[pl.ds(0, _WCH)], sem).wait()

    def body(t, _):
        for par in range(2):
            tl = 2 * t + par
            c = wid + _NW * tl
            cn = wid + _NW * (tl + 1)
            cprev = wid + _NW * (tl - 2)

            @pl.when(jnp.logical_and(tl > 1, cprev < n_chunks))
            def _():
                drain_out(vouts[par], sos[par])

            @pl.when(c < n_chunks)
            def _():
                if par == 0:
                    @pl.when(tl == 0)
                    def _():
                        stage_in(c, vins[par], sis[par])

                @pl.when(cn < n_chunks)
                def _():
                    stage_in(cn, vins[par ^ 1], sis[par ^ 1])

                drain_in(vins[par], sis[par])
                fill(vins[par], vouts[par])
                stage_out(c, vouts[par], sos[par])

        return 0

    n_steps = 2 * ((max_t + 1) // 2)
    lax.fori_loop(0, n_steps // 2, body, 0)
    for par in range(2):
        c_last = wid + _NW * (n_steps - 2 + par)

        @pl.when(c_last < n_chunks)
        def _():
            drain_out(vouts[par], sos[par])


def _gather_kernel(batches_per_w, hist, idx_hbm, wide_hbm, out_hbm,
                   idxb0, idxb1, rows0, rows1,
                   si0, si1, sg0, sg1, so0, so1):
    wid = lax.axis_index("s") * 2 + lax.axis_index("c")
    b_base = wid * batches_per_w

    splits = [(0, 128), (128, hist - 128)] if hist > 128 else [(0, hist)]

    def stage_idx(g, idxb, sem):
        pltpu.async_copy(idx_hbm.at[b_base + g], idxb, sem)

    def drain_idx(idxb, sem):
        pltpu.make_async_copy(idx_hbm.at[b_base], idxb, sem).wait()

    def fire_group(idxb, rows, sem):
        for (off, ln) in splits:
            pltpu.async_copy(
                wide_hbm.at[idxb.at[pl.ds(off, ln)]],
                rows.at[pl.ds(off, ln)],
                sem)

    def fire_out(g, rows, sem):
        pltpu.async_copy(rows, out_hbm.at[b_base + g], sem)

    def drain_gather(rows, sem):
        pltpu.make_async_copy(
            wide_hbm.at[pl.ds(0, hist)], rows, sem).wait()

    def drain_out(rows, sem):
        pltpu.make_async_copy(rows, out_hbm.at[b_base], sem).wait()

    pltpu.sync_copy(idx_hbm.at[b_base], idxb0)
    fire_group(idxb0, rows0, sg0)
    stage_idx(1, idxb1, si1)
    npairs = batches_per_w // 2

    def body(t, _):
        a = 2 * t
        more = t < npairs - 1

        @pl.when(t > 0)
        def _():
            drain_out(rows1, so1)

        drain_idx(idxb1, si1)
        fire_group(idxb1, rows1, sg1)
        drain_gather(rows0, sg0)

        @pl.when(more)
        def _():
            stage_idx(a + 2, idxb0, si0)

        fire_out(a, rows0, so0)
        drain_out(rows0, so0)

        @pl.when(more)
        def _():
            drain_idx(idxb0, si0)
            fire_group(idxb0, rows0, sg0)

        drain_gather(rows1, sg1)

        @pl.when(more)
        def _():
            stage_idx(a + 3, idxb1, si1)

        fire_out(a + 1, rows1, so1)
        return 0

    lax.fori_loop(0, npairs, body, 0)
    drain_out(rows1, so1)


def kernel(paragraph_variable, embedding_table):
    B, H = paragraph_variable.shape
    V, D = embedding_table.shape
    batches_per_w = B // _NW
    n_chunks = V // _WCH

    mesh = plsc.VectorSubcoreMesh(core_axis_name="c", subcore_axis_name="s")

    widen = pl.kernel(
        functools.partial(_widen_kernel, n_chunks),
        mesh=mesh,
        out_type=jax.ShapeDtypeStruct((V, 2 * D), jnp.float32),
        scratch_types=[
            pltpu.VMEM((_WCH, D), jnp.float32),
            pltpu.VMEM((_WCH, D), jnp.float32),
            pltpu.VMEM((_WCH, 2 * D), jnp.float32),
            pltpu.VMEM((_WCH, 2 * D), jnp.float32),
            pltpu.SemaphoreType.DMA,
            pltpu.SemaphoreType.DMA,
            pltpu.SemaphoreType.DMA,
            pltpu.SemaphoreType.DMA,
        ],
    )
    wide = widen(embedding_table)

    gather = pl.kernel(
        functools.partial(_gather_kernel, batches_per_w, H),
        mesh=mesh,
        out_type=jax.ShapeDtypeStruct((B, H, 2 * D), jnp.float32),
        scratch_types=[
            pltpu.VMEM((H,), jnp.int32),
            pltpu.VMEM((H,), jnp.int32),
            pltpu.VMEM((H, 2 * D), jnp.float32),
            pltpu.VMEM((H, 2 * D), jnp.float32),
            pltpu.SemaphoreType.DMA,
            pltpu.SemaphoreType.DMA,
            pltpu.SemaphoreType.DMA,
            pltpu.SemaphoreType.DMA,
            pltpu.SemaphoreType.DMA,
            pltpu.SemaphoreType.DMA,
        ],
    )
    res = gather(paragraph_variable, wide)
    return res[:, :, :D]


# speedup vs baseline: 1.1646x; 1.1646x over previous
import functools

import jax
import jax.numpy as jnp
from jax import lax
from jax.experimental import pallas as pl
from jax.experimental.pallas import tpu as pltpu
from jax.experimental.pallas import tpu_sc as plsc

_NW = 32


def _gather_kernel(batches_per_w, hist, idx_hbm, wide_hbm, out_hbm,
                   idx_v, rows0, rows1, sg0, sg1, so0, so1):
    wid = lax.axis_index("s") * 2 + lax.axis_index("c")
    b_base = wid * batches_per_w
    pltpu.sync_copy(idx_hbm.at[pl.ds(b_base, batches_per_w)], idx_v)

    splits = [(0, 128), (128, hist - 128)] if hist > 128 else [(0, hist)]

    def fire_group(g, rows, sem):
        for (off, ln) in splits:
            pltpu.async_copy(
                wide_hbm.at[idx_v.at[g, pl.ds(off, ln)]],
                rows.at[pl.ds(off, ln)],
                sem)

    def fire_out(g, rows, sem):
        pltpu.async_copy(rows, out_hbm.at[b_base + g], sem)

    def drain_gather(rows, sem):
        pltpu.make_async_copy(
            wide_hbm.at[pl.ds(0, hist)], rows, sem).wait()

    def drain_out(rows, sem):
        pltpu.make_async_copy(rows, out_hbm.at[b_base], sem).wait()

    fire_group(0, rows0, sg0)
    npairs = batches_per_w // 2

    def body(t, _):
        a = 2 * t

        @pl.when(t > 0)
        def _():
            drain_out(rows1, so1)

        fire_group(a + 1, rows1, sg1)
        drain_gather(rows0, sg0)
        fire_out(a, rows0, so0)
        drain_out(rows0, so0)

        @pl.when(t < npairs - 1)
        def _():
            fire_group(a + 2, rows0, sg0)

        drain_gather(rows1, sg1)
        fire_out(a + 1, rows1, so1)
        return 0

    lax.fori_loop(0, npairs, body, 0)
    drain_out(rows1, so1)


def kernel(paragraph_variable, embedding_table):
    B, H = paragraph_variable.shape
    V, D = embedding_table.shape
    batches_per_w = B // _NW

    wide = lax.dynamic_update_slice(
        jnp.zeros((V, 2 * D), jnp.float32), embedding_table, (0, 0))

    mesh = plsc.VectorSubcoreMesh(core_axis_name="c", subcore_axis_name="s")
    gather = pl.kernel(
        functools.partial(_gather_kernel, batches_per_w, H),
        mesh=mesh,
        out_type=jax.ShapeDtypeStruct((B, H, 2 * D), jnp.float32),
        scratch_types=[
            pltpu.VMEM((batches_per_w, H), jnp.int32),
            pltpu.VMEM((H, 2 * D), jnp.float32),
            pltpu.VMEM((H, 2 * D), jnp.float32),
            pltpu.SemaphoreType.DMA,
            pltpu.SemaphoreType.DMA,
            pltpu.SemaphoreType.DMA,
            pltpu.SemaphoreType.DMA,
        ],
    )
    res = gather(paragraph_variable, wide)
    return res[:, :, :D]
